# baseline (speedup 1.0000x reference)
import functools

import jax
import jax.numpy as jnp
import numpy as np
from jax import lax
from jax.experimental import pallas as pl
from jax.experimental.pallas import tpu as pltpu
from jax.experimental.pallas import tpu_sc as plsc

N_ROWS = 64
N_COLS = 32768
K_RATIO = 0.1

_POS = np.float32(1.0 - K_RATIO) * np.float32(N_COLS - 1)
_LO_IDX = int(np.floor(_POS))
_GAMMA = float(np.float32(_POS) - np.float32(_LO_IDX))
_RANK = _LO_IDX + 1

_MAX_BITS = 0x7F7FFFFF

_NVEC = N_COLS // 16
_WORDS_PER_ROW = N_COLS // 4


def _locate(counts, rank, iota):
    def body(j, carry):
        found, bsel, below, c_at = carry
        c = counts[pl.ds(16 * j, 16)]
        tot = jnp.sum(c)
        cum = plsc.cumsum(c)
        hitv = jnp.logical_and(below + cum >= rank, found == 0)
        nhit = jnp.max(plsc.all_reduce_population_count(hitv))
        lane = jnp.max(plsc.all_reduce_ffs(hitv))
        is_hit = jnp.logical_and(found == 0, nhit > 0)
        cumprev = jnp.sum(jnp.where(iota < lane, c, 0))
        c_lane = jnp.sum(jnp.where(iota == lane, c, 0))
        bsel = jnp.where(is_hit, 16 * j + lane, bsel)
        below = jnp.where(is_hit, below + cumprev,
                          jnp.where(found == 0, below + tot, below))
        c_at = jnp.where(is_hit, c_lane, c_at)
        found = jnp.where(is_hit, 1, found)
        return found, bsel, below, c_at

    zero = jnp.int32(0)
    _, b, below, c_at = lax.fori_loop(0, 64, body, (zero, zero, zero, zero))
    return b, below, c_at


def _row_tau(row, cand, counts, iota, ones):
    def zero_counts():
        @plsc.parallel_loop(0, 64, unroll=4)
        def _(j):
            counts[pl.ds(16 * j, 16)] = jnp.zeros((16,), jnp.int32)

    zero_counts()

    @plsc.parallel_loop(0, _NVEC, unroll=8)
    def _(i):
        k = plsc.bitcast(row[pl.ds(16 * i, 16)], jnp.int32)
        plsc.addupdate_scatter(counts, [jnp.right_shift(k, 20)], ones)

    rank = jnp.int32(_RANK)
    b0, below0, c0 = _locate(counts, rank, iota)
    rank = rank - below0

    b0v = jnp.full((16,), 1, jnp.int32) * b0
    hi_lim = jnp.left_shift(b0v + 1, 20)
    maxv = jnp.full((16,), _MAX_BITS, jnp.int32)

    def compact_body(i, c):
        off, hi_min = c
        k = plsc.bitcast(row[pl.ds(16 * i, 16)], jnp.int32)
        m = jnp.right_shift(k, 20) == b0v
        plsc.store_compressed(cand.at[pl.ds(off, 16)], k, mask=m)
        npop = jnp.max(plsc.all_reduce_population_count(m))
        hi_min = jnp.minimum(hi_min, jnp.where(k >= hi_lim, k, maxv))
        return off + npop, hi_min

    c0_off, hi_min = lax.fori_loop(
        0, _NVEC, compact_body,
        (jnp.int32(0), jnp.full((16,), _MAX_BITS, jnp.int32)))
    cand[pl.ds(c0_off, 16)] = maxv
    nc = jnp.right_shift(c0_off + 15, 4)

    def cand_hist(shift, prefix_shift, prefix):
        pv = jnp.full((16,), 1, jnp.int32) * prefix

        def body(i, _):
            k = cand[pl.ds(16 * i, 16)]
            bucket = jnp.bitwise_and(jnp.right_shift(k, shift), 1023)
            m = jnp.right_shift(k, prefix_shift) == pv
            plsc.addupdate_scatter(counts, [bucket], ones, mask=m)
            return 0

        lax.fori_loop(0, nc, body, 0)

    zero_counts()
    cand_hist(10, 20, b0)
    b1, below1, _ = _locate(counts, rank, iota)
    rank = rank - below1

    zero_counts()
    prefix01 = jnp.bitwise_or(jnp.left_shift(b0, 10), b1)
    cand_hist(0, 10, prefix01)
    b2, below2, c_eq = _locate(counts, rank, iota)
    rank = rank - below2

    s1_bits = jnp.bitwise_or(jnp.left_shift(prefix01, 10), b2)

    s1v = jnp.full((16,), 1, jnp.int32) * s1_bits

    def succ_body(i, a):
        k = cand[pl.ds(16 * i, 16)]
        return jnp.minimum(a, jnp.where(k > s1v, k, maxv))

    acc = lax.fori_loop(0, nc, succ_body, hi_min)
    nxt_bits = jnp.min(acc)
    s2_bits = jnp.where(rank < c_eq, s1_bits, nxt_bits)

    s1 = plsc.bitcast(jnp.full((16,), 1, jnp.int32) * s1_bits, jnp.float32)
    s2 = plsc.bitcast(jnp.full((16,), 1, jnp.int32) * s2_bits, jnp.float32)
    g = jnp.float32(_GAMMA)
    return s1 * (jnp.float32(1) - g) + s2 * g


def _sc_kernel(a_hbm, b_hbm, newp_hbm, disp_hbm,
               row_a, row_b, cand, counts, mask_n, mask_d):
    wid = lax.axis_index("s") * 2 + lax.axis_index("c")
    iota = lax.iota(jnp.int32, 16)
    ones = jnp.full((16,), 1, jnp.int32)
    offs = [4 * iota + j for j in range(4)]

    def do_row(rr, _):
        r = 2 * wid + rr
        base = r * N_COLS
        pltpu.sync_copy(a_hbm.at[pl.ds(base, N_COLS)], row_a)
        pltpu.sync_copy(b_hbm.at[pl.ds(base, N_COLS)], row_b)

        tau_a = _row_tau(row_a, cand, counts, iota, ones)
        tau_b = _row_tau(row_b, cand, counts, iota, ones)

        @plsc.parallel_loop(0, _NVEC // 4, unroll=4)
        def mask_body(i):
            gbase = 64 * i
            nw = jnp.zeros((16,), jnp.int32)
            dw = jnp.zeros((16,), jnp.int32)
            for j in range(4):
                idx = gbase + offs[j]
                av = plsc.load_gather(row_a, [idx])
                bv = plsc.load_gather(row_b, [idx])
                a_hi = av > tau_a
                b_hi = bv > tau_b
                nm = jnp.where(jnp.logical_and(a_hi, jnp.logical_not(b_hi)),
                               1, 0)
                dm = jnp.where(jnp.logical_and(b_hi, jnp.logical_not(a_hi)),
                               1, 0)
                nw = jnp.bitwise_or(nw, jnp.left_shift(nm, 8 * j))
                dw = jnp.bitwise_or(dw, jnp.left_shift(dm, 8 * j))
            mask_n[pl.ds(16 * i, 16)] = nw
            mask_d[pl.ds(16 * i, 16)] = dw

        obase = r * _WORDS_PER_ROW
        pltpu.sync_copy(mask_n, newp_hbm.at[pl.ds(obase, _WORDS_PER_ROW)])
        pltpu.sync_copy(mask_d, disp_hbm.at[pl.ds(obase, _WORDS_PER_ROW)])
        return 0

    lax.fori_loop(0, 2, do_row, 0)


@jax.jit
def kernel(C_now2past, C_past2now):
    rows, cols = C_now2past.shape
    nwords = rows * cols // 4
    mesh = plsc.VectorSubcoreMesh(core_axis_name="c", subcore_axis_name="s")
    packed_struct = jax.ShapeDtypeStruct((nwords,), jnp.int32)
    run = functools.partial(
        pl.kernel,
        mesh=mesh,
        out_type=[packed_struct, packed_struct],
        compiler_params=pltpu.CompilerParams(needs_layout_passes=False),
        scratch_types=[
            pltpu.VMEM((N_COLS,), jnp.float32),
            pltpu.VMEM((N_COLS,), jnp.float32),
            pltpu.VMEM((N_COLS + 16,), jnp.int32),
            pltpu.VMEM((1024,), jnp.int32),
            pltpu.VMEM((N_COLS // 4,), jnp.int32),
            pltpu.VMEM((N_COLS // 4,), jnp.int32),
        ],
    )(_sc_kernel)
    newp, disp = run(C_now2past.reshape(-1), C_past2now.reshape(-1))

    def unpack(p):
        bytes4 = lax.bitcast_convert_type(p.reshape(rows, cols // 4),
                                          jnp.uint8)
        return bytes4.reshape(rows, cols).astype(jnp.bool_)

    return (unpack(newp), unpack(disp))


# speedup vs baseline: 1.3859x; 1.3859x over previous
import functools

import jax
import jax.numpy as jnp
import numpy as np
from jax import lax
from jax.experimental import pallas as pl
from jax.experimental.pallas import tpu as pltpu
from jax.experimental.pallas import tpu_sc as plsc

N_ROWS = 64
N_COLS = 32768
K_RATIO = 0.1

_POS = np.float32(1.0 - K_RATIO) * np.float32(N_COLS - 1)
_LO_IDX = int(np.floor(_POS))
_GAMMA = float(np.float32(_POS) - np.float32(_LO_IDX))
_RANK = _LO_IDX + 1

_MAX_BITS = 0x7F7FFFFF

_NVEC = N_COLS // 16
_WORDS_PER_ROW = N_COLS // 4


def _locate(counts, rank, iota):
    def body(j, carry):
        found, bsel, below, c_at = carry
        c = counts[pl.ds(16 * j, 16)]
        tot = jnp.sum(c)
        cum = plsc.cumsum(c)
        hitv = jnp.logical_and(below + cum >= rank, found == 0)
        lane = jnp.max(plsc.all_reduce_ffs(hitv))
        is_hit = jnp.logical_and(found == 0, lane < 16)
        cumprev = jnp.sum(jnp.where(iota < lane, c, 0))
        c_lane = jnp.sum(jnp.where(iota == lane, c, 0))
        bsel = jnp.where(is_hit, 16 * j + lane, bsel)
        below = jnp.where(is_hit, below + cumprev,
                          jnp.where(found == 0, below + tot, below))
        c_at = jnp.where(is_hit, c_lane, c_at)
        found = jnp.where(is_hit, 1, found)
        return found, bsel, below, c_at

    zero = jnp.int32(0)
    _, b, below, c_at = lax.fori_loop(0, 64, body, (zero, zero, zero, zero))
    return b, below, c_at


def _row_tau(row, counts, iota, ones):
    def zero_counts():
        @plsc.parallel_loop(0, 64, unroll=4)
        def _(j):
            counts[pl.ds(16 * j, 16)] = jnp.zeros((16,), jnp.int32)

    def hist(shift, prefix_shift, prefix):
        pv = jnp.full((16,), 1, jnp.int32) * prefix

        @plsc.parallel_loop(0, _NVEC, unroll=16)
        def _(i):
            k = plsc.bitcast(row[pl.ds(16 * i, 16)], jnp.int32)
            bucket = jnp.bitwise_and(jnp.right_shift(k, shift), 1023)
            if prefix_shift >= 30:
                plsc.addupdate_scatter(counts, [bucket], ones)
            else:
                m = jnp.right_shift(k, prefix_shift) == pv
                plsc.addupdate_scatter(counts, [bucket], ones, mask=m)

    rank = jnp.int32(_RANK)
    zero_counts()
    hist(20, 30, jnp.int32(0))
    b0, below0, _ = _locate(counts, rank, iota)
    rank = rank - below0

    zero_counts()
    hist(10, 20, b0)
    b1, below1, _ = _locate(counts, rank, iota)
    rank = rank - below1

    zero_counts()
    prefix01 = jnp.bitwise_or(jnp.left_shift(b0, 10), b1)
    hist(0, 10, prefix01)
    b2, below2, c_eq = _locate(counts, rank, iota)
    rank = rank - below2

    s1_bits = jnp.bitwise_or(jnp.left_shift(prefix01, 10), b2)

    s1v = jnp.full((16,), 1, jnp.int32) * s1_bits
    maxv = jnp.full((16,), _MAX_BITS, jnp.int32)

    @plsc.parallel_loop(0, _NVEC, unroll=16,
                        carry=jnp.full((16,), _MAX_BITS, jnp.int32))
    def acc(i, a):
        k = plsc.bitcast(row[pl.ds(16 * i, 16)], jnp.int32)
        return jnp.minimum(a, jnp.where(k > s1v, k, maxv))

    nxt_bits = jnp.min(acc)
    s2_bits = jnp.where(rank < c_eq, s1_bits, nxt_bits)

    s1 = plsc.bitcast(jnp.full((16,), 1, jnp.int32) * s1_bits, jnp.float32)
    s2 = plsc.bitcast(jnp.full((16,), 1, jnp.int32) * s2_bits, jnp.float32)
    g = jnp.float32(_GAMMA)
    return s1 * (jnp.float32(1) - g) + s2 * g


def _sc_kernel(a_hbm, b_hbm, newp_hbm, disp_hbm,
               row_a, row_b, counts, mask_n, mask_d):
    wid = lax.axis_index("s") * 2 + lax.axis_index("c")
    iota = lax.iota(jnp.int32, 16)
    ones = jnp.full((16,), 1, jnp.int32)
    offs = [4 * iota + j for j in range(4)]

    def do_row(rr, _):
        r = 2 * wid + rr
        base = r * N_COLS
        pltpu.sync_copy(a_hbm.at[pl.ds(base, N_COLS)], row_a)
        pltpu.sync_copy(b_hbm.at[pl.ds(base, N_COLS)], row_b)

        tau_a = _row_tau(row_a, counts, iota, ones)
        tau_b = _row_tau(row_b, counts, iota, ones)

        @plsc.parallel_loop(0, _NVEC // 4, unroll=4)
        def mask_body(i):
            gbase = 64 * i
            nw = jnp.zeros((16,), jnp.int32)
            dw = jnp.zeros((16,), jnp.int32)
            for j in range(4):
                idx = gbase + offs[j]
                av = plsc.load_gather(row_a, [idx])
                bv = plsc.load_gather(row_b, [idx])
                a_hi = av > tau_a
                b_hi = bv > tau_b
                nm = jnp.where(jnp.logical_and(a_hi, jnp.logical_not(b_hi)),
                               1, 0)
                dm = jnp.where(jnp.logical_and(b_hi, jnp.logical_not(a_hi)),
                               1, 0)
                nw = jnp.bitwise_or(nw, jnp.left_shift(nm, 8 * j))
                dw = jnp.bitwise_or(dw, jnp.left_shift(dm, 8 * j))
            mask_n[pl.ds(16 * i, 16)] = nw
            mask_d[pl.ds(16 * i, 16)] = dw

        obase = r * _WORDS_PER_ROW
        pltpu.sync_copy(mask_n, newp_hbm.at[pl.ds(obase, _WORDS_PER_ROW)])
        pltpu.sync_copy(mask_d, disp_hbm.at[pl.ds(obase, _WORDS_PER_ROW)])
        return 0

    lax.fori_loop(0, 2, do_row, 0)


@jax.jit
def kernel(C_now2past, C_past2now):
    rows, cols = C_now2past.shape
    nwords = rows * cols // 4
    mesh = plsc.VectorSubcoreMesh(core_axis_name="c", subcore_axis_name="s")
    packed_struct = jax.ShapeDtypeStruct((nwords,), jnp.int32)
    run = functools.partial(
        pl.kernel,
        mesh=mesh,
        out_type=[packed_struct, packed_struct],
        compiler_params=pltpu.CompilerParams(needs_layout_passes=False),
        scratch_types=[
            pltpu.VMEM((N_COLS,), jnp.float32),
            pltpu.VMEM((N_COLS,), jnp.float32),
            pltpu.VMEM((1024,), jnp.int32),
            pltpu.VMEM((N_COLS // 4,), jnp.int32),
            pltpu.VMEM((N_COLS // 4,), jnp.int32),
        ],
    )(_sc_kernel)
    newp, disp = run(C_now2past.reshape(-1), C_past2now.reshape(-1))

    def unpack(p):
        bytes4 = lax.bitcast_convert_type(p.reshape(rows, cols // 4),
                                          jnp.uint8)
        return bytes4.reshape(rows, cols).astype(jnp.bool_)

    return (unpack(newp), unpack(disp))
